# baseline (speedup 1.0000x reference)
import functools

import jax
import jax.numpy as jnp
from jax import lax
from jax.experimental import pallas as pl
from jax.experimental.pallas import tpu as pltpu
from jax.experimental.pallas import tpu_sc as plsc

BS, NA, D = 2, 900, 256
C, H, W = 256, 200, 200
HW = H * W
L = 16
NC, NS = 2, 16
NW = NC * NS
BPAD = 1024
NPAD = BS * BPAD
CPW = C // NW
NG = NPAD // L
NGB = BPAD // L

XMIN, XMAX = -80.0, 120.0
YMIN, YMAX = -40.0, 40.0

_mesh = plsc.VectorSubcoreMesh(
    core_axis_name="c", subcore_axis_name="s", num_cores=NC, num_subcores=NS)


def _floor_i(x):
    t = x.astype(jnp.int32)
    tf = t.astype(jnp.float32)
    t = t - jnp.where(tf > x, 1, 0)
    return t, t.astype(jnp.float32)


@functools.partial(
    pl.kernel,
    out_type=jax.ShapeDtypeStruct((C, NPAD), jnp.float32),
    mesh=_mesh,
    scratch_types=[
        pltpu.VMEM((NPAD,), jnp.float32),
        pltpu.VMEM((NPAD,), jnp.float32),
        pltpu.VMEM((4 * NPAD,), jnp.int32),
        pltpu.VMEM((4 * NPAD,), jnp.float32),
        pltpu.VMEM((HW,), jnp.float32),
        pltpu.VMEM((HW,), jnp.float32),
        pltpu.VMEM((BPAD,), jnp.float32),
        pltpu.SemaphoreType.DMA,
        pltpu.SemaphoreType.DMA,
    ],
    compiler_params=pltpu.CompilerParams(needs_layout_passes=False),
)
def _sc_bilinear(ax_hbm, ay_hbm, fm_hbm, out_hbm,
                 ax_v, ay_v, idx_v, w_v, buf0, buf1, frow, sem0, sem1):
    wid = lax.axis_index("s") * NC + lax.axis_index("c")
    c0 = wid * CPW

    pltpu.async_copy(fm_hbm.at[c0], buf0, sem0)
    pltpu.async_copy(fm_hbm.at[C + c0], buf1, sem1)

    pltpu.sync_copy(ax_hbm, ax_v)
    pltpu.sync_copy(ay_hbm, ay_v)

    @plsc.parallel_loop(0, NG, unroll=2)
    def _prelude(g):
        x = ax_v[pl.ds(g * L, L)]
        y = ay_v[pl.ds(g * L, L)]
        gx = (y - YMIN) / (YMAX - YMIN + 1e-06) * 2.0 - 1.0
        gy = (x - XMIN) / (XMAX - XMIN + 1e-06) * 2.0 - 1.0
        ix = (gx + 1.0) * 0.5 * (W - 1)
        iy = (gy + 1.0) * 0.5 * (H - 1)
        ix = jnp.clip(ix, -4.0, W + 4.0)
        iy = jnp.clip(iy, -4.0, H + 4.0)
        x0, x0f = _floor_i(ix)
        y0, y0f = _floor_i(iy)
        dx0 = ix - x0f
        dx1 = (x0f + 1.0) - ix
        dy0 = iy - y0f
        dy1 = (y0f + 1.0) - iy
        vx0 = (x0 >= 0) & (x0 < W)
        vx1 = (x0 >= -1) & (x0 < W - 1)
        vy0 = (y0 >= 0) & (y0 < H)
        vy1 = (y0 >= -1) & (y0 < H - 1)
        xc0 = jnp.clip(x0, 0, W - 1)
        xc1 = jnp.clip(x0 + 1, 0, W - 1)
        yo0 = jnp.clip(y0, 0, H - 1) * W
        yo1 = jnp.clip(y0 + 1, 0, H - 1) * W
        corners = (
            (yo0 + xc0, vy0 & vx0, dx1 * dy1),
            (yo1 + xc0, vy1 & vx0, dx1 * dy0),
            (yo0 + xc1, vy0 & vx1, dx0 * dy1),
            (yo1 + xc1, vy1 & vx1, dx0 * dy0),
        )
        for k, (pix, valid, wgt) in enumerate(corners):
            idx_v[pl.ds(k * NPAD + g * L, L)] = pix
            w_v[pl.ds(k * NPAD + g * L, L)] = jnp.where(valid, wgt, 0.0)

    @pl.loop(0, CPW)
    def _planes(ci):
        c = c0 + ci
        for b, buf, sem in ((0, buf0, sem0), (1, buf1, sem1)):
            pltpu.make_async_copy(fm_hbm.at[0], buf, sem).wait()

            @plsc.parallel_loop(0, NGB, unroll=4)
            def _acc(g):
                s = b * BPAD + g * L
                v00 = plsc.load_gather(buf, [idx_v[pl.ds(0 * NPAD + s, L)]])
                v10 = plsc.load_gather(buf, [idx_v[pl.ds(1 * NPAD + s, L)]])
                v01 = plsc.load_gather(buf, [idx_v[pl.ds(2 * NPAD + s, L)]])
                v11 = plsc.load_gather(buf, [idx_v[pl.ds(3 * NPAD + s, L)]])
                frow[pl.ds(g * L, L)] = (
                    w_v[pl.ds(0 * NPAD + s, L)] * v00
                    + w_v[pl.ds(1 * NPAD + s, L)] * v10
                    + w_v[pl.ds(2 * NPAD + s, L)] * v01
                    + w_v[pl.ds(3 * NPAD + s, L)] * v11)

            @pl.when(ci < CPW - 1)
            def _start_next():
                pltpu.async_copy(fm_hbm.at[b * C + c + 1], buf, sem)

            pltpu.sync_copy(frow, out_hbm.at[c, pl.ds(b * BPAD, BPAD)])


def _mm_body(ft_ref, w_ref, b_ref, inst_ref, o_ref):
    o_ref[...] = (
        lax.dot_general(ft_ref[...], w_ref[...], (((0,), (1,)), ((), ())),
                        preferred_element_type=jnp.float32)
        + b_ref[...] + inst_ref[...])


def _tc_proj(featsT, w_proj, b2, inst):
    return pl.pallas_call(
        _mm_body,
        out_shape=jax.ShapeDtypeStruct((NPAD, D), jnp.float32),
    )(featsT, w_proj, b2, inst)


def kernel(instance_feature, anchor, anchor_embed, feature_maps, W_proj, b_proj):
    ax = jnp.pad(anchor[..., 0], ((0, 0), (0, BPAD - NA))).reshape(-1)
    ay = jnp.pad(anchor[..., 1], ((0, 0), (0, BPAD - NA))).reshape(-1)
    fm2d = feature_maps.reshape(BS * C, HW)
    featsT = _sc_bilinear(ax, ay, fm2d)
    inst = jnp.pad(instance_feature, ((0, 0), (0, BPAD - NA), (0, 0)))
    out = _tc_proj(featsT, W_proj, b_proj.reshape(1, D), inst.reshape(NPAD, D))
    return out.reshape(BS, BPAD, D)[:, :NA, :]
